# baseline (speedup 1.0000x reference)
import jax
import jax.numpy as jnp
from jax import lax
from jax.experimental import pallas as pl
from jax.experimental.pallas import tpu as pltpu

F32 = jnp.float32
BF16 = jnp.bfloat16
B, C, T = 16, 256, 256
NB = 8
W = NB * T


def _body(maxlen_ref, lbs_ref, x_ref, dur_ref, pt_ref, et_ref, w6, vecs,
          out_ref, mel_ref, dpred_ref, ppred_ref, epred_ref):
    row_i = lax.broadcasted_iota(jnp.int32, (C, C), 0).astype(F32)
    col_i = lax.broadcasted_iota(jnp.int32, (C, C), 1).astype(F32)
    upper = (row_i <= col_i).astype(BF16)
    pcol = lax.broadcasted_iota(jnp.int32, (C, 1), 0).astype(F32)
    maxlen_f = maxlen_ref[0].astype(F32)
    ones_col = jnp.full((C, 1), 1.0, BF16)

    colw = lax.broadcasted_iota(jnp.int32, (1, W), 1)
    tmod = jnp.bitwise_and(colw, T - 1)
    mask_first = (tmod != 0).astype(BF16)
    mask_last = (tmod != T - 1).astype(BF16)

    rW = lax.broadcasted_iota(jnp.int32, (W, NB), 0)
    cW = lax.broadcasted_iota(jnp.int32, (W, NB), 1)
    seg = (rW // T) == cW
    bd01 = seg.astype(F32)
    bd = bd01 * (1.0 / T)
    rWt = lax.broadcasted_iota(jnp.int32, (NB, W), 0)
    cWt = lax.broadcasted_iota(jnp.int32, (NB, W), 1)
    bdt = ((cWt // T) == rWt).astype(F32)

    def vrow(i):
        return vecs[i:i + 1, :]

    def vtile(i):
        r = vrow(i)
        return jnp.concatenate([r] * NB, axis=1)

    def vcolT(i):
        return jnp.transpose(vrow(i))

    def shifts(xb):
        xm = jnp.concatenate([jnp.zeros((C, 1), BF16), xb[:, :-1]],
                             axis=1) * mask_first
        xp = jnp.concatenate([xb[:, 1:], jnp.zeros((C, 1), BF16)],
                             axis=1) * mask_last
        return xm, xp

    def conv_big(xb, wi, bcol):
        w = w6[wi]
        xm, xp = shifts(xb)
        a = jnp.dot(w[0:C, :], xm, preferred_element_type=F32)
        a = a + jnp.dot(w[C:2 * C, :], xb, preferred_element_type=F32)
        a = a + jnp.dot(w[2 * C:3 * C, :], xp, preferred_element_type=F32)
        return a + bcol

    def ln_big(h, gbig, bebig):
        mu_s = jnp.dot(h, bd, preferred_element_type=F32)
        mu = jnp.dot(mu_s, bdt, preferred_element_type=F32)
        hc = h - mu
        var_s = jnp.dot(hc * hc, bd, preferred_element_type=F32)
        r = lax.rsqrt(var_s + 1e-5)
        rb = jnp.dot(r, bdt, preferred_element_type=F32)
        return hc * rb * gbig + bebig

    def vp_big(xb, wi, v0, lb):
        h = jnp.maximum(conv_big(xb, wi, vcolT(v0)), 0.0)
        h = ln_big(h, vtile(v0 + 1), vtile(v0 + 2))
        h2 = jnp.maximum(conv_big(h.astype(BF16), wi + 1, vcolT(v0 + 3)), 0.0)
        h2 = ln_big(h2, vtile(v0 + 4), vtile(v0 + 5))
        pred = jnp.dot(h2 * vtile(v0 + 6), bd01,
                       preferred_element_type=F32)
        return jnp.transpose(pred + lb)

    lb_d = lbs_ref[0]
    lb_p = lbs_ref[1]
    lb_e = lbs_ref[2]

    xparts = [x_ref[i].astype(BF16) for i in range(NB)]
    xbig = jnp.concatenate(xparts, axis=1)

    dpred_ref[0] = vp_big(xbig, 0, 0, lb_d)

    dmat = dur_ref[...].astype(BF16)
    cs = jnp.dot(dmat, upper, preferred_element_type=F32)
    totals = cs[:, T - 1:T]
    parts2 = []
    for b in range(NB):
        cs_b = cs[b:b + 1, :]
        cmp = (row_i >= cs_b).astype(BF16)
        idx = jnp.dot(cmp, ones_col, preferred_element_type=F32)
        total_b = totals[b:b + 1, 0:1]
        valid = (pcol < total_b) & (pcol < maxlen_f)
        onehot = ((idx == col_i) & valid).astype(BF16)
        parts2.append(jnp.dot(onehot, xparts[b], preferred_element_type=F32))
    mel_ref[0] = totals.astype(jnp.int32)

    x2 = jnp.concatenate(parts2, axis=1)
    x2b = x2.astype(BF16)

    ppred_ref[0] = vp_big(x2b, 2, 7, lb_p)
    epred_ref[0] = vp_big(x2b, 4, 14, lb_e)

    ptcols = jnp.transpose(pt_ref[...])
    etcols = jnp.transpose(et_ref[...])
    for b in range(NB):
        pc = ptcols[:, b:b + 1]
        ec = etcols[:, b:b + 1]
        pcm = jnp.concatenate([jnp.zeros((1, 1), F32), pc[:-1, :]], axis=0)
        pcp = jnp.concatenate([pc[1:, :], jnp.zeros((1, 1), F32)], axis=0)
        ecm = jnp.concatenate([jnp.zeros((1, 1), F32), ec[:-1, :]], axis=0)
        ecp = jnp.concatenate([ec[1:, :], jnp.zeros((1, 1), F32)], axis=0)
        emb = (pcm * vrow(21) + pc * vrow(22) + pcp * vrow(23) + vrow(27)
               + ecm * vrow(24) + ec * vrow(25) + ecp * vrow(26) + vrow(28))
        out_ref[b] = parts2[b] + emb


def _full(shape):
    nd = len(shape)
    return pl.BlockSpec(shape, lambda b: (0,) * nd)


def kernel(x, src_len, duration_target, pitch_target, energy_target, max_len,
           dp_w1, dp_b1, dp_g1, dp_be1, dp_w2, dp_b2, dp_g2, dp_be2, dp_lw, dp_lb,
           pp_w1, pp_b1, pp_g1, pp_be1, pp_w2, pp_b2, pp_g2, pp_be2, pp_lw, pp_lb,
           ep_w1, ep_b1, ep_g1, ep_be1, ep_w2, ep_b2, ep_g2, ep_be2, ep_lw, ep_lb,
           pe_w, pe_b, ee_w, ee_b):
    del src_len
    maxlen = jnp.asarray(max_len, jnp.int32).reshape(1)
    lbs = jnp.concatenate([dp_lb, pp_lb, ep_lb]).astype(F32)

    w6 = jnp.stack([
        jnp.transpose(w, (2, 0, 1)).reshape(3 * C, C)
        for w in (dp_w1, dp_w2, pp_w1, pp_w2, ep_w1, ep_w2)
    ]).astype(BF16)

    vecs = jnp.stack([
        dp_b1, dp_g1, dp_be1, dp_b2, dp_g2, dp_be2, dp_lw.reshape(T),
        pp_b1, pp_g1, pp_be1, pp_b2, pp_g2, pp_be2, pp_lw.reshape(T),
        ep_b1, ep_g1, ep_be1, ep_b2, ep_g2, ep_be2, ep_lw.reshape(T),
        pe_w[:, 0, 0], pe_w[:, 0, 1], pe_w[:, 0, 2],
        ee_w[:, 0, 0], ee_w[:, 0, 1], ee_w[:, 0, 2],
        pe_b, ee_b,
    ]).astype(F32)

    in_specs = [pl.BlockSpec(memory_space=pltpu.SMEM),
                pl.BlockSpec(memory_space=pltpu.SMEM),
                pl.BlockSpec((NB, C, T), lambda s: (s, 0, 0)),
                pl.BlockSpec((NB, T), lambda s: (s, 0)),
                pl.BlockSpec((NB, T), lambda s: (s, 0)),
                pl.BlockSpec((NB, T), lambda s: (s, 0)),
                _full((6, 3 * C, C)),
                _full((29, T))]

    out_shapes = (
        jax.ShapeDtypeStruct((B, C, T), F32),
        jax.ShapeDtypeStruct((B // NB, NB, 1), jnp.int32),
        jax.ShapeDtypeStruct((B // NB, NB, C), F32),
        jax.ShapeDtypeStruct((B // NB, NB, C), F32),
        jax.ShapeDtypeStruct((B // NB, NB, C), F32),
    )
    out_specs = (
        pl.BlockSpec((NB, C, T), lambda s: (s, 0, 0)),
        pl.BlockSpec((1, NB, 1), lambda s: (s, 0, 0)),
        pl.BlockSpec((1, NB, C), lambda s: (s, 0, 0)),
        pl.BlockSpec((1, NB, C), lambda s: (s, 0, 0)),
        pl.BlockSpec((1, NB, C), lambda s: (s, 0, 0)),
    )

    out, mel, dpred, ppred, epred = pl.pallas_call(
        _body,
        grid=(B // NB,),
        in_specs=in_specs,
        out_specs=out_specs,
        out_shape=out_shapes,
        compiler_params=pltpu.CompilerParams(
            dimension_semantics=("parallel",)),
    )(maxlen, lbs, x, duration_target.astype(jnp.int32), pitch_target,
      energy_target, w6, vecs)

    return (out, mel.reshape(B), dpred.reshape(B, C), ppred.reshape(B, C),
            epred.reshape(B, C))
